# baseline (speedup 1.0000x reference)
import functools

import numpy as np
import jax
import jax.numpy as jnp
from jax import lax
from jax.experimental import pallas as pl
from jax.experimental.pallas import tpu as pltpu
from jax.experimental.pallas import tpu_sc as plsc

N = 10000
E = 320000
D = 128
P = 10000

NC = 2
NS = 16
NW = NC * NS
EPW = E // NW
GW = 80
NG = EPW // GW
NPAD = 10240
SLAB = NPAD // NS

_M = N // 2
_T = 25
_BL = _M // _T


def _hilbert_tables() -> np.ndarray:
    m = np.arange(N)
    with np.errstate(divide="ignore"):
        t = np.where(m % 2 == 1, -(2.0 / N) / np.tan(np.pi * np.maximum(m, 1) / N), 0.0)
    t[0] = 0.0
    p = np.arange(_M)
    u_eo = t[(2 * p - 1) % N]
    u_oe = t[(2 * p + 1) % N]
    k = np.arange(2 * _M)[:, None]
    r = np.arange(_BL)[None, :]
    idx = (r - k) % _M
    return np.stack([u_eo[idx], u_oe[idx]])


_VSTACK = _hilbert_tables()


@functools.cache
def _sc_mesh():
    return plsc.VectorSubcoreMesh(
        core_axis_name="c", subcore_axis_name="s", num_cores=NC, num_subcores=NS)


NB = 8
NBUF = 4
EPW2 = 10240
NG2 = EPW2 // GW
NFULL = NG2 // NB


def _sc_aggregate_body(feats, src3, dst3, zrows,
                       out, isrc8, idst8, rows, acc, sem_g, sem_s):
    cid = lax.axis_index("c")
    sid = lax.axis_index("s")
    wid = cid * NS + sid
    pltpu.sync_copy(zrows, acc.at[pl.ds(sid * SLAB, SLAB)])
    plsc.subcore_barrier()

    def block(i, carry):
        pltpu.sync_copy(src3.at[wid, pl.ds(i * NB, NB)], isrc8)
        pltpu.sync_copy(dst3.at[wid, pl.ds(i * NB, NB)], idst8)
        gd = []
        sd = []
        for b in range(NB):
            if b >= NBUF:
                sd[b - NBUF].wait()
            gd.append(pltpu.async_copy(
                feats.at[isrc8.at[b]], rows.at[b % NBUF], sem_g))
            if b >= NBUF - 1:
                j = b - (NBUF - 1)
                gd[j].wait()
                sd.append(pltpu.async_copy(
                    rows.at[j % NBUF], acc.at[idst8.at[j]], sem_s, add=True))
        for j in range(NB - NBUF + 1, NB):
            gd[j].wait()
            sd.append(pltpu.async_copy(
                rows.at[j % NBUF], acc.at[idst8.at[j]], sem_s, add=True))
        for d in sd[NB - NBUF:]:
            d.wait()
        return carry

    lax.fori_loop(0, NFULL, block, 0)
    plsc.subcore_barrier()
    pltpu.sync_copy(
        acc.at[pl.ds(sid * SLAB, SLAB)],
        out.at[cid, pl.ds(sid * SLAB, SLAB)],
    )


@functools.cache
def _sc_aggregate_kernel():
    return pl.kernel(
        _sc_aggregate_body,
        out_type=jax.ShapeDtypeStruct((NC, NPAD, D), jnp.float32),
        mesh=_sc_mesh(),
        scratch_types=[
            pltpu.VMEM((NB, GW), jnp.int32),
            pltpu.VMEM((NB, GW), jnp.int32),
            pltpu.VMEM((NBUF, GW, D), jnp.float32),
            pltpu.VMEM_SHARED((NPAD, D), jnp.float32),
            pltpu.SemaphoreType.DMA,
            pltpu.SemaphoreType.DMA,
        ],
    )


def _sc_aggregate(feats_pad, src3, dst3, zrows):
    return _sc_aggregate_kernel()(feats_pad, src3, dst3, zrows)


def _sc_degree_body(dst3, zrows, ones_in, out, idst8, ones_v, acc, sem_s):
    cid = lax.axis_index("c")
    sid = lax.axis_index("s")
    wid = cid * NS + sid
    pltpu.sync_copy(zrows, acc.at[pl.ds(sid * SLAB, SLAB)])
    pltpu.sync_copy(ones_in, ones_v)
    plsc.subcore_barrier()

    def block(i, carry):
        pltpu.sync_copy(dst3.at[wid, pl.ds(i * NB, NB)], idst8)
        sd = [pltpu.async_copy(ones_v, acc.at[idst8.at[b]], sem_s, add=True)
              for b in range(NB)]
        for d in sd:
            d.wait()
        return carry

    lax.fori_loop(0, NFULL, block, 0)
    plsc.subcore_barrier()
    pltpu.sync_copy(
        acc.at[pl.ds(sid * SLAB, SLAB)],
        out.at[cid, pl.ds(sid * SLAB, SLAB)],
    )


@functools.cache
def _sc_degree_kernel():
    return pl.kernel(
        _sc_degree_body,
        out_type=jax.ShapeDtypeStruct((NC, NPAD, D), jnp.float32),
        mesh=_sc_mesh(),
        scratch_types=[
            pltpu.VMEM((NB, GW), jnp.int32),
            pltpu.VMEM((GW, D), jnp.float32),
            pltpu.VMEM_SHARED((NPAD, D), jnp.float32),
            pltpu.SemaphoreType.DMA,
        ],
    )


def _sc_degree(dst3, zrows, ones_in):
    return _sc_degree_kernel()(dst3, zrows, ones_in)


_PPAD = 10240
_PPW = _PPAD // NW
_PL = _PPW // 16


def _sc_decode_body(pred, eli0, eli1, out, pred_v, e0, e1, prod):
    cid = lax.axis_index("c")
    sid = lax.axis_index("s")
    wid = cid * NS + sid
    pltpu.sync_copy(pred, pred_v)
    pltpu.sync_copy(eli0.at[pl.ds(wid * _PPW, _PPW)], e0)
    pltpu.sync_copy(eli1.at[pl.ds(wid * _PPW, _PPW)], e1)
    for l in range(_PL):
        n0 = e0[pl.ds(l * 16, 16)]
        n1 = e1[pl.ds(l * 16, 16)]
        f0 = (n0 & 1) * _M + (n0 >> 1)
        f1 = (n1 & 1) * _M + (n1 >> 1)
        a = plsc.load_gather(pred_v, [f0])
        b = plsc.load_gather(pred_v, [f1])
        prod[pl.ds(l * 16, 16)] = a * b
    pltpu.sync_copy(prod, out.at[pl.ds(wid * _PPW, _PPW)])


@functools.cache
def _sc_decode_kernel():
    return pl.kernel(
        _sc_decode_body,
        out_type=jax.ShapeDtypeStruct((_PPAD,), jnp.float32),
        mesh=_sc_mesh(),
        scratch_types=[
            pltpu.VMEM((N,), jnp.float32),
            pltpu.VMEM((_PPW,), jnp.int32),
            pltpu.VMEM((_PPW,), jnp.int32),
            pltpu.VMEM((_PPW,), jnp.float32),
        ],
        compiler_params=pltpu.CompilerParams(needs_layout_passes=False),
    )


def _sc_decode(pred_flat, eli0, eli1):
    return _sc_decode_kernel()(pred_flat, eli0, eli1)


def _tc_prep_body(x_ref, degp_ref, xs_ref, dinv_ref):
    deg = degp_ref[0, :N, :] + degp_ref[1, :N, :]
    dinv = jnp.where(deg > 0.0, lax.rsqrt(deg), 0.0)
    dinv_ref[...] = dinv[:, :16]
    xs_ref[pl.ds(0, N), :] = x_ref[...] * dinv[:, 0:1]
    xs_ref[pl.ds(N, NPAD - N), :] = jnp.zeros((NPAD - N, D), jnp.float32)


def _tc_prep(x, degp):
    return pl.pallas_call(
        _tc_prep_body,
        out_shape=(
            jax.ShapeDtypeStruct((NPAD, D), jnp.float32),
            jax.ShapeDtypeStruct((N, 16), jnp.float32),
        ),
    )(x, degp)


def _tc_layer_body(aggp_ref, dinv_ref, skip_ref, mw_ref, mb_ref, sw_ref, sb_ref,
                   h_ref, xs_ref, *, want_xs):
    dinv = dinv_ref[:, 0:1]
    rst = (aggp_ref[0, :N, :] + aggp_ref[1, :N, :]) * dinv
    skip_in = skip_ref[...]
    h = (
        lax.dot_general(rst, mw_ref[...], (((1,), (1,)), ((), ())),
                        preferred_element_type=jnp.float32)
        + mb_ref[...]
        + lax.dot_general(skip_in, sw_ref[...], (((1,), (1,)), ((), ())),
                          preferred_element_type=jnp.float32)
        + sb_ref[...]
    )
    h_ref[...] = h
    if want_xs:
        xs_ref[pl.ds(0, N), :] = h * dinv
        xs_ref[pl.ds(N, NPAD - N), :] = jnp.zeros((NPAD - N, D), jnp.float32)


def _tc_layer(aggp, dinv, skip_in, mw, mb, sw, sb, want_xs):
    outs = [jax.ShapeDtypeStruct((N, D), jnp.float32)]
    if want_xs:
        outs.append(jax.ShapeDtypeStruct((NPAD, D), jnp.float32))
        body = functools.partial(_tc_layer_body, want_xs=True)
    else:
        def body(aggp_ref, dinv_ref, skip_ref, mw_ref, mb_ref, sw_ref, sb_ref, h_ref):
            _tc_layer_body(aggp_ref, dinv_ref, skip_ref, mw_ref, mb_ref, sw_ref,
                           sb_ref, h_ref, None, want_xs=False)
    return pl.pallas_call(body, out_shape=tuple(outs))(
        aggp, dinv, skip_in, mw, mb.reshape(1, D), sw, sb.reshape(1, D))


def _tc_filter_mlp_body(v_ref, rhs_ref, skip_ref, cwt_ref, w1_ref, w2_ref, out_ref):
    i = pl.program_id(1)
    lhs_t = v_ref[0, pl.ds(pl.multiple_of(_M - i * _BL, 8), _M), :]
    rhs = rhs_ref[0]
    acc = lax.dot_general(lhs_t, rhs, (((0,), (0,)), ((), ())),
                          preferred_element_type=jnp.float32)
    skip = skip_ref[0]
    a_row = cwt_ref[0:1, :]
    b_row = cwt_ref[1:2, :]
    h2 = skip * (1.0 + a_row) + acc * b_row
    nrm = jnp.maximum(jnp.sqrt(jnp.sum(h2 * h2, axis=1, keepdims=True)), 1e-12)
    hn = h2 / nrm
    r = jnp.maximum(
        lax.dot_general(hn, w1_ref[...], (((1,), (1,)), ((), ())),
                        preferred_element_type=jnp.float32), 0.0)
    pred = jax.nn.sigmoid(
        lax.dot_general(r, w2_ref[...], (((1,), (1,)), ((), ())),
                        preferred_element_type=jnp.float32))
    out_ref[0] = jnp.broadcast_to(pred, (_BL, 16))


def _tc_filter_mlp(h1, cwt, w1, w2):
    hpar = h1.reshape(_M, 2, D).transpose(1, 0, 2)
    hpar_bf = hpar.astype(jnp.bfloat16)
    vstack = jnp.asarray(_VSTACK, dtype=jnp.bfloat16)
    grid = (2, _T)
    out = pl.pallas_call(
        _tc_filter_mlp_body,
        grid=grid,
        in_specs=[
            pl.BlockSpec((1, 2 * _M, _BL), lambda j, i: (j, 0, 0)),
            pl.BlockSpec((1, _M, D), lambda j, i: (1 - j, 0, 0)),
            pl.BlockSpec((1, _BL, D), lambda j, i: (j, i, 0)),
            pl.BlockSpec((2, D), lambda j, i: (0, 0)),
            pl.BlockSpec((D, D), lambda j, i: (0, 0)),
            pl.BlockSpec((1, D), lambda j, i: (0, 0)),
        ],
        out_specs=pl.BlockSpec((1, _BL, 16), lambda j, i: (j, i, 0)),
        out_shape=jax.ShapeDtypeStruct((2, _M, 16), jnp.float32),
    )(vstack, hpar_bf, hpar, cwt, w1, w2)
    return out[:, :, 0].reshape(2 * _M)


def kernel(x, edge_index, edge_label_index, weight1, weight2,
           skip_w0, skip_b0, msg_w0, msg_b0,
           skip_w1, skip_b1, msg_w1, msg_b1, complex_weight):
    pad = jnp.full((NW, EPW2 - EPW), N, jnp.int32)
    src3 = jnp.concatenate(
        [edge_index[0].reshape(NW, EPW), pad], axis=1).reshape(NW, NG2, GW)
    dst3 = jnp.concatenate(
        [edge_index[1].reshape(NW, EPW), pad], axis=1).reshape(NW, NG2, GW)
    zrows = jnp.zeros((SLAB, D), jnp.float32)
    ones_in = jnp.ones((GW, D), jnp.float32)

    degp = _sc_degree(src3, zrows, ones_in)
    xs0, dinv = _tc_prep(x, degp)

    agg0 = _sc_aggregate(xs0, src3, dst3, zrows)
    h0, xs1 = _tc_layer(agg0, dinv, x, msg_w0, msg_b0, skip_w0, skip_b0, True)

    agg1 = _sc_aggregate(xs1, src3, dst3, zrows)
    (h1,) = _tc_layer(agg1, dinv, h0, msg_w1, msg_b1, skip_w1, skip_b1, False)

    pred_flat = _tc_filter_mlp(h1, complex_weight.T, weight1, weight2)

    eli0 = jnp.pad(edge_label_index[0], (0, _PPAD - P))
    eli1 = jnp.pad(edge_label_index[1], (0, _PPAD - P))
    prod = _sc_decode(pred_flat, eli0, eli1)
    return prod[:P]


# speedup vs baseline: 2.1824x; 2.1824x over previous
import functools

import numpy as np
import jax
import jax.numpy as jnp
from jax import lax
from jax.experimental import pallas as pl
from jax.experimental.pallas import tpu as pltpu
from jax.experimental.pallas import tpu_sc as plsc

N = 10000
E = 320000
D = 128
P = 10000

NC = 2
NS = 16
NW = NC * NS
EPW = E // NW
GW = 80
NG = EPW // GW
NPAD = 10240
SLAB = NPAD // NS

_M = N // 2
_T = 25
_BL = _M // _T


def _hilbert_tables() -> np.ndarray:
    m = np.arange(N)
    with np.errstate(divide="ignore"):
        t = np.where(m % 2 == 1, -(2.0 / N) / np.tan(np.pi * np.maximum(m, 1) / N), 0.0)
    t[0] = 0.0
    p = np.arange(_M)
    u_eo = t[(2 * p - 1) % N]
    u_oe = t[(2 * p + 1) % N]
    k = np.arange(2 * _M)[:, None]
    r = np.arange(_BL)[None, :]
    idx = (r - k) % _M
    return np.stack([u_eo[idx], u_oe[idx]])


_VSTACK = _hilbert_tables()


@functools.cache
def _sc_mesh():
    return plsc.VectorSubcoreMesh(
        core_axis_name="c", subcore_axis_name="s", num_cores=NC, num_subcores=NS)


NB = 8
NBUF = 4
EPW2 = 10240
NG2 = EPW2 // GW
NFULL = NG2 // NB


def _sc_aggregate_body(feats, src3, dst3, zrows,
                       out, isrc8, idst8, rows, acc, sem_g, sem_s):
    cid = lax.axis_index("c")
    sid = lax.axis_index("s")
    wid = cid * NS + sid
    pltpu.sync_copy(zrows, acc.at[pl.ds(sid * SLAB, SLAB)])
    plsc.subcore_barrier()

    def block(i, carry):
        pltpu.sync_copy(src3.at[wid, pl.ds(i * NB, NB)], isrc8)
        pltpu.sync_copy(dst3.at[wid, pl.ds(i * NB, NB)], idst8)
        gd = []
        sd = []
        for b in range(NB):
            if b >= NBUF:
                sd[b - NBUF].wait()
            gd.append(pltpu.async_copy(
                feats.at[isrc8.at[b]], rows.at[b % NBUF], sem_g))
            if b >= NBUF - 1:
                j = b - (NBUF - 1)
                gd[j].wait()
                sd.append(pltpu.async_copy(
                    rows.at[j % NBUF], acc.at[idst8.at[j]], sem_s, add=True))
        for j in range(NB - NBUF + 1, NB):
            gd[j].wait()
            sd.append(pltpu.async_copy(
                rows.at[j % NBUF], acc.at[idst8.at[j]], sem_s, add=True))
        for d in sd[NB - NBUF:]:
            d.wait()
        return carry

    lax.fori_loop(0, NFULL, block, 0)
    plsc.subcore_barrier()
    pltpu.sync_copy(
        acc.at[pl.ds(sid * SLAB, SLAB)],
        out.at[cid, pl.ds(sid * SLAB, SLAB)],
    )


@functools.cache
def _sc_aggregate_kernel():
    return pl.kernel(
        _sc_aggregate_body,
        out_type=jax.ShapeDtypeStruct((NC, NPAD, D), jnp.float32),
        mesh=_sc_mesh(),
        scratch_types=[
            pltpu.VMEM((NB, GW), jnp.int32),
            pltpu.VMEM((NB, GW), jnp.int32),
            pltpu.VMEM((NBUF, GW, D), jnp.float32),
            pltpu.VMEM_SHARED((NPAD, D), jnp.float32),
            pltpu.SemaphoreType.DMA,
            pltpu.SemaphoreType.DMA,
        ],
    )


def _sc_aggregate(feats_pad, src3, dst3, zrows):
    return _sc_aggregate_kernel()(feats_pad, src3, dst3, zrows)


def _sc_degree_body(dst3, zrows, ones_in, out, idst8, ones_v, acc, sem_s):
    cid = lax.axis_index("c")
    sid = lax.axis_index("s")
    wid = cid * NS + sid
    pltpu.sync_copy(zrows, acc.at[pl.ds(sid * SLAB, SLAB)])
    pltpu.sync_copy(ones_in, ones_v)
    plsc.subcore_barrier()

    def block(i, carry):
        pltpu.sync_copy(dst3.at[wid, pl.ds(i * NB, NB)], idst8)
        sd = [pltpu.async_copy(ones_v, acc.at[idst8.at[b]], sem_s, add=True)
              for b in range(NB)]
        for d in sd:
            d.wait()
        return carry

    lax.fori_loop(0, NFULL, block, 0)
    plsc.subcore_barrier()
    pltpu.sync_copy(
        acc.at[pl.ds(sid * SLAB, SLAB)],
        out.at[cid, pl.ds(sid * SLAB, SLAB)],
    )


@functools.cache
def _sc_degree_kernel():
    return pl.kernel(
        _sc_degree_body,
        out_type=jax.ShapeDtypeStruct((NC, NPAD, D), jnp.float32),
        mesh=_sc_mesh(),
        scratch_types=[
            pltpu.VMEM((NB, GW), jnp.int32),
            pltpu.VMEM((GW, D), jnp.float32),
            pltpu.VMEM_SHARED((NPAD, D), jnp.float32),
            pltpu.SemaphoreType.DMA,
        ],
    )


def _sc_degree(dst3, zrows, ones_in):
    return _sc_degree_kernel()(dst3, zrows, ones_in)


_PPAD = 10240
_PPW = _PPAD // NW
_PL = _PPW // 16


def _sc_decode_body(pred, eli0, eli1, out, pred_v, e0, e1, prod):
    cid = lax.axis_index("c")
    sid = lax.axis_index("s")
    wid = cid * NS + sid
    pltpu.sync_copy(pred, pred_v)
    pltpu.sync_copy(eli0.at[pl.ds(wid * _PPW, _PPW)], e0)
    pltpu.sync_copy(eli1.at[pl.ds(wid * _PPW, _PPW)], e1)
    for l in range(_PL):
        n0 = e0[pl.ds(l * 16, 16)]
        n1 = e1[pl.ds(l * 16, 16)]
        f0 = (n0 & 1) * _M + (n0 >> 1)
        f1 = (n1 & 1) * _M + (n1 >> 1)
        a = plsc.load_gather(pred_v, [f0])
        b = plsc.load_gather(pred_v, [f1])
        prod[pl.ds(l * 16, 16)] = a * b
    pltpu.sync_copy(prod, out.at[pl.ds(wid * _PPW, _PPW)])


@functools.cache
def _sc_decode_kernel():
    return pl.kernel(
        _sc_decode_body,
        out_type=jax.ShapeDtypeStruct((_PPAD,), jnp.float32),
        mesh=_sc_mesh(),
        scratch_types=[
            pltpu.VMEM((N,), jnp.float32),
            pltpu.VMEM((_PPW,), jnp.int32),
            pltpu.VMEM((_PPW,), jnp.int32),
            pltpu.VMEM((_PPW,), jnp.float32),
        ],
        compiler_params=pltpu.CompilerParams(needs_layout_passes=False),
    )


def _sc_decode(pred_flat, eli0, eli1):
    return _sc_decode_kernel()(pred_flat, eli0, eli1)


def _tc_prep_body(x_ref, degp_ref, xs_ref, dinv_ref):
    deg = degp_ref[0, :N, :] + degp_ref[1, :N, :]
    dinv = jnp.where(deg > 0.0, lax.rsqrt(deg), 0.0)
    dinv_ref[...] = dinv[:, :16]
    xs_ref[pl.ds(0, N), :] = x_ref[...] * dinv[:, 0:1]
    xs_ref[pl.ds(N, NPAD - N), :] = jnp.zeros((NPAD - N, D), jnp.float32)


def _tc_prep(x, degp):
    return pl.pallas_call(
        _tc_prep_body,
        out_shape=(
            jax.ShapeDtypeStruct((NPAD, D), jnp.float32),
            jax.ShapeDtypeStruct((N, 16), jnp.float32),
        ),
    )(x, degp)


def _tc_layer_body(aggp_ref, dinv_ref, skip_ref, mw_ref, mb_ref, sw_ref, sb_ref,
                   h_ref, xs_ref, *, want_xs):
    dinv = dinv_ref[:, 0:1]
    rst = (aggp_ref[0, :N, :] + aggp_ref[1, :N, :]) * dinv
    skip_in = skip_ref[...]
    h = (
        lax.dot_general(rst, mw_ref[...], (((1,), (1,)), ((), ())),
                        preferred_element_type=jnp.float32)
        + mb_ref[...]
        + lax.dot_general(skip_in, sw_ref[...], (((1,), (1,)), ((), ())),
                          preferred_element_type=jnp.float32)
        + sb_ref[...]
    )
    h_ref[...] = h
    if want_xs:
        xs_ref[pl.ds(0, N), :] = h * dinv
        xs_ref[pl.ds(N, NPAD - N), :] = jnp.zeros((NPAD - N, D), jnp.float32)


def _tc_layer(aggp, dinv, skip_in, mw, mb, sw, sb, want_xs):
    outs = [jax.ShapeDtypeStruct((N, D), jnp.float32)]
    if want_xs:
        outs.append(jax.ShapeDtypeStruct((NPAD, D), jnp.float32))
        body = functools.partial(_tc_layer_body, want_xs=True)
    else:
        def body(aggp_ref, dinv_ref, skip_ref, mw_ref, mb_ref, sw_ref, sb_ref, h_ref):
            _tc_layer_body(aggp_ref, dinv_ref, skip_ref, mw_ref, mb_ref, sw_ref,
                           sb_ref, h_ref, None, want_xs=False)
    return pl.pallas_call(body, out_shape=tuple(outs))(
        aggp, dinv, skip_in, mw, mb.reshape(1, D), sw, sb.reshape(1, D))


def _tc_filter_mlp_body(v_ref, rhs_ref, skip_ref, cwt_ref, w1_ref, w2_ref, out_ref):
    i = pl.program_id(1)
    lhs_t = v_ref[0, pl.ds(pl.multiple_of(_M - i * _BL, 8), _M), :]
    rhs = rhs_ref[0]
    acc = lax.dot_general(lhs_t, rhs, (((0,), (0,)), ((), ())),
                          preferred_element_type=jnp.float32)
    skip = skip_ref[0]
    a_row = cwt_ref[0:1, :]
    b_row = cwt_ref[1:2, :]
    h2 = skip * (1.0 + a_row) + acc * b_row
    nrm = jnp.maximum(jnp.sqrt(jnp.sum(h2 * h2, axis=1, keepdims=True)), 1e-12)
    hn = h2 / nrm
    r = jnp.maximum(
        lax.dot_general(hn, w1_ref[...], (((1,), (1,)), ((), ())),
                        preferred_element_type=jnp.float32), 0.0)
    pred = jax.nn.sigmoid(
        lax.dot_general(r, w2_ref[...], (((1,), (1,)), ((), ())),
                        preferred_element_type=jnp.float32))
    out_ref[0] = jnp.broadcast_to(pred, (_BL, 16))


def _tc_filter_mlp(h1, cwt, w1, w2):
    hpar = h1.reshape(_M, 2, D).transpose(1, 0, 2)
    hpar_bf = hpar.astype(jnp.bfloat16)
    vstack = jnp.asarray(_VSTACK, dtype=jnp.bfloat16)
    grid = (2, _T)
    out = pl.pallas_call(
        _tc_filter_mlp_body,
        grid=grid,
        in_specs=[
            pl.BlockSpec((1, 2 * _M, _BL), lambda j, i: (j, 0, 0)),
            pl.BlockSpec((1, _M, D), lambda j, i: (1 - j, 0, 0)),
            pl.BlockSpec((1, _BL, D), lambda j, i: (j, i, 0)),
            pl.BlockSpec((2, D), lambda j, i: (0, 0)),
            pl.BlockSpec((D, D), lambda j, i: (0, 0)),
            pl.BlockSpec((1, D), lambda j, i: (0, 0)),
        ],
        out_specs=pl.BlockSpec((1, _BL, 16), lambda j, i: (j, i, 0)),
        out_shape=jax.ShapeDtypeStruct((2, _M, 16), jnp.float32),
    )(vstack, hpar_bf, hpar, cwt, w1, w2)
    return out[:, :, 0].reshape(2 * _M)


def kernel(x, edge_index, edge_label_index, weight1, weight2,
           skip_w0, skip_b0, msg_w0, msg_b0,
           skip_w1, skip_b1, msg_w1, msg_b1, complex_weight):
    pad = jnp.broadcast_to(
        N + jnp.arange(EPW2 - EPW, dtype=jnp.int32), (NW, EPW2 - EPW))
    src3 = jnp.concatenate(
        [edge_index[0].reshape(NW, EPW), pad], axis=1).reshape(NW, NG2, GW)
    dst3 = jnp.concatenate(
        [edge_index[1].reshape(NW, EPW), pad], axis=1).reshape(NW, NG2, GW)
    zrows = jnp.zeros((SLAB, D), jnp.float32)
    ones_in = jnp.ones((GW, D), jnp.float32)

    degp = _sc_degree(src3, zrows, ones_in)
    xs0, dinv = _tc_prep(x, degp)

    agg0 = _sc_aggregate(xs0, src3, dst3, zrows)
    h0, xs1 = _tc_layer(agg0, dinv, x, msg_w0, msg_b0, skip_w0, skip_b0, True)

    agg1 = _sc_aggregate(xs1, src3, dst3, zrows)
    (h1,) = _tc_layer(agg1, dinv, h0, msg_w1, msg_b1, skip_w1, skip_b1, False)

    pred_flat = _tc_filter_mlp(h1, complex_weight.T, weight1, weight2)

    eli0 = jnp.pad(edge_label_index[0], (0, _PPAD - P))
    eli1 = jnp.pad(edge_label_index[1], (0, _PPAD - P))
    prod = _sc_decode(pred_flat, eli0, eli1)
    return prod[:P]


# speedup vs baseline: 2.2784x; 1.0440x over previous
import functools

import numpy as np
import jax
import jax.numpy as jnp
from jax import lax
from jax.experimental import pallas as pl
from jax.experimental.pallas import tpu as pltpu
from jax.experimental.pallas import tpu_sc as plsc

N = 10000
E = 320000
D = 128
P = 10000

NC = 2
NS = 16
NW = NC * NS
EPW = E // NW
GW = 80
NG = EPW // GW
NPAD = 10240
SLAB = NPAD // NS

_M = N // 2
_T = 5
_BL = _M // _T


def _hilbert_tables() -> np.ndarray:
    m = np.arange(N)
    with np.errstate(divide="ignore"):
        t = np.where(m % 2 == 1, -(2.0 / N) / np.tan(np.pi * np.maximum(m, 1) / N), 0.0)
    t[0] = 0.0
    p = np.arange(_M)
    u_eo = t[(2 * p - 1) % N]
    u_oe = t[(2 * p + 1) % N]
    k = np.arange(2 * _M)[:, None]
    r = np.arange(_BL)[None, :]
    idx = (r - k) % _M
    return np.stack([u_eo[idx], u_oe[idx]])


_VSTACK = _hilbert_tables()


@functools.cache
def _sc_mesh():
    return plsc.VectorSubcoreMesh(
        core_axis_name="c", subcore_axis_name="s", num_cores=NC, num_subcores=NS)


NB = 8
NBUF = 4
EPW2 = 10240
NG2 = EPW2 // GW
NFULL = NG2 // NB


def _sc_aggregate_body(feats, src3, dst3, zrows,
                       out, isrc8, idst8, rows, acc, sem_g, sem_s):
    cid = lax.axis_index("c")
    sid = lax.axis_index("s")
    wid = cid * NS + sid
    pltpu.sync_copy(zrows, acc.at[pl.ds(sid * SLAB, SLAB)])
    plsc.subcore_barrier()

    def block(i, carry):
        pltpu.sync_copy(src3.at[wid, pl.ds(i * NB, NB)], isrc8)
        pltpu.sync_copy(dst3.at[wid, pl.ds(i * NB, NB)], idst8)
        gd = []
        sd = []
        for b in range(NB):
            if b >= NBUF:
                sd[b - NBUF].wait()
            gd.append(pltpu.async_copy(
                feats.at[isrc8.at[b]], rows.at[b % NBUF], sem_g))
            if b >= NBUF - 1:
                j = b - (NBUF - 1)
                gd[j].wait()
                sd.append(pltpu.async_copy(
                    rows.at[j % NBUF], acc.at[idst8.at[j]], sem_s, add=True))
        for j in range(NB - NBUF + 1, NB):
            gd[j].wait()
            sd.append(pltpu.async_copy(
                rows.at[j % NBUF], acc.at[idst8.at[j]], sem_s, add=True))
        for d in sd[NB - NBUF:]:
            d.wait()
        return carry

    lax.fori_loop(0, NFULL, block, 0)
    plsc.subcore_barrier()
    pltpu.sync_copy(
        acc.at[pl.ds(sid * SLAB, SLAB)],
        out.at[cid, pl.ds(sid * SLAB, SLAB)],
    )


@functools.cache
def _sc_aggregate_kernel():
    return pl.kernel(
        _sc_aggregate_body,
        out_type=jax.ShapeDtypeStruct((NC, NPAD, D), jnp.float32),
        mesh=_sc_mesh(),
        scratch_types=[
            pltpu.VMEM((NB, GW), jnp.int32),
            pltpu.VMEM((NB, GW), jnp.int32),
            pltpu.VMEM((NBUF, GW, D), jnp.float32),
            pltpu.VMEM_SHARED((NPAD, D), jnp.float32),
            pltpu.SemaphoreType.DMA,
            pltpu.SemaphoreType.DMA,
        ],
    )


def _sc_aggregate(feats_pad, src3, dst3, zrows):
    return _sc_aggregate_kernel()(feats_pad, src3, dst3, zrows)


def _sc_degree_body(dst3, zrows, ones_in, out, idst8, ones_v, acc, sem_s):
    cid = lax.axis_index("c")
    sid = lax.axis_index("s")
    wid = cid * NS + sid
    pltpu.sync_copy(zrows, acc.at[pl.ds(sid * SLAB, SLAB)])
    pltpu.sync_copy(ones_in, ones_v)
    plsc.subcore_barrier()

    def block(i, carry):
        pltpu.sync_copy(dst3.at[wid, pl.ds(i * NB, NB)], idst8)
        sd = [pltpu.async_copy(ones_v, acc.at[idst8.at[b]], sem_s, add=True)
              for b in range(NB)]
        for d in sd:
            d.wait()
        return carry

    lax.fori_loop(0, NFULL, block, 0)
    plsc.subcore_barrier()
    pltpu.sync_copy(
        acc.at[pl.ds(sid * SLAB, SLAB)],
        out.at[cid, pl.ds(sid * SLAB, SLAB)],
    )


@functools.cache
def _sc_degree_kernel():
    return pl.kernel(
        _sc_degree_body,
        out_type=jax.ShapeDtypeStruct((NC, NPAD, D), jnp.float32),
        mesh=_sc_mesh(),
        scratch_types=[
            pltpu.VMEM((NB, GW), jnp.int32),
            pltpu.VMEM((GW, D), jnp.float32),
            pltpu.VMEM_SHARED((NPAD, D), jnp.float32),
            pltpu.SemaphoreType.DMA,
        ],
    )


def _sc_degree(dst3, zrows, ones_in):
    return _sc_degree_kernel()(dst3, zrows, ones_in)


_PPAD = 10240
_PPW = _PPAD // NW
_PL = _PPW // 16


def _sc_decode_body(pred, eli0, eli1, out, pred_v, e0, e1, prod):
    cid = lax.axis_index("c")
    sid = lax.axis_index("s")
    wid = cid * NS + sid
    pltpu.sync_copy(pred, pred_v)
    pltpu.sync_copy(eli0.at[pl.ds(wid * _PPW, _PPW)], e0)
    pltpu.sync_copy(eli1.at[pl.ds(wid * _PPW, _PPW)], e1)
    for l in range(_PL):
        n0 = e0[pl.ds(l * 16, 16)]
        n1 = e1[pl.ds(l * 16, 16)]
        f0 = (n0 & 1) * _M + (n0 >> 1)
        f1 = (n1 & 1) * _M + (n1 >> 1)
        a = plsc.load_gather(pred_v, [f0])
        b = plsc.load_gather(pred_v, [f1])
        prod[pl.ds(l * 16, 16)] = a * b
    pltpu.sync_copy(prod, out.at[pl.ds(wid * _PPW, _PPW)])


@functools.cache
def _sc_decode_kernel():
    return pl.kernel(
        _sc_decode_body,
        out_type=jax.ShapeDtypeStruct((_PPAD,), jnp.float32),
        mesh=_sc_mesh(),
        scratch_types=[
            pltpu.VMEM((N,), jnp.float32),
            pltpu.VMEM((_PPW,), jnp.int32),
            pltpu.VMEM((_PPW,), jnp.int32),
            pltpu.VMEM((_PPW,), jnp.float32),
        ],
        compiler_params=pltpu.CompilerParams(needs_layout_passes=False),
    )


def _sc_decode(pred_flat, eli0, eli1):
    return _sc_decode_kernel()(pred_flat, eli0, eli1)


def _tc_prep_body(x_ref, degp_ref, xs_ref, dinv_ref):
    deg = degp_ref[0, :N, :] + degp_ref[1, :N, :]
    dinv = jnp.where(deg > 0.0, lax.rsqrt(deg), 0.0)
    dinv_ref[...] = dinv[:, :16]
    xs_ref[pl.ds(0, N), :] = x_ref[...] * dinv[:, 0:1]
    xs_ref[pl.ds(N, NPAD - N), :] = jnp.zeros((NPAD - N, D), jnp.float32)


def _tc_prep(x, degp):
    return pl.pallas_call(
        _tc_prep_body,
        out_shape=(
            jax.ShapeDtypeStruct((NPAD, D), jnp.float32),
            jax.ShapeDtypeStruct((N, 16), jnp.float32),
        ),
    )(x, degp)


def _tc_layer_body(aggp_ref, dinv_ref, skip_ref, mw_ref, mb_ref, sw_ref, sb_ref,
                   h_ref, xs_ref, *, want_xs):
    dinv = dinv_ref[:, 0:1]
    rst = (aggp_ref[0, :N, :] + aggp_ref[1, :N, :]) * dinv
    skip_in = skip_ref[...]
    h = (
        lax.dot_general(rst, mw_ref[...], (((1,), (1,)), ((), ())),
                        preferred_element_type=jnp.float32)
        + mb_ref[...]
        + lax.dot_general(skip_in, sw_ref[...], (((1,), (1,)), ((), ())),
                          preferred_element_type=jnp.float32)
        + sb_ref[...]
    )
    h_ref[...] = h
    if want_xs:
        xs_ref[pl.ds(0, N), :] = h * dinv
        xs_ref[pl.ds(N, NPAD - N), :] = jnp.zeros((NPAD - N, D), jnp.float32)


def _tc_layer(aggp, dinv, skip_in, mw, mb, sw, sb, want_xs):
    outs = [jax.ShapeDtypeStruct((N, D), jnp.float32)]
    if want_xs:
        outs.append(jax.ShapeDtypeStruct((NPAD, D), jnp.float32))
        body = functools.partial(_tc_layer_body, want_xs=True)
    else:
        def body(aggp_ref, dinv_ref, skip_ref, mw_ref, mb_ref, sw_ref, sb_ref, h_ref):
            _tc_layer_body(aggp_ref, dinv_ref, skip_ref, mw_ref, mb_ref, sw_ref,
                           sb_ref, h_ref, None, want_xs=False)
    return pl.pallas_call(body, out_shape=tuple(outs))(
        aggp, dinv, skip_in, mw, mb.reshape(1, D), sw, sb.reshape(1, D))


def _tc_filter_mlp_body(v_ref, rhs_ref, skip_ref, cwt_ref, w1_ref, w2_ref, out_ref):
    i = pl.program_id(1)
    lhs_t = v_ref[0, pl.ds(pl.multiple_of(_M - i * _BL, 8), _M), :]
    rhs = rhs_ref[0]
    acc = lax.dot_general(lhs_t, rhs, (((0,), (0,)), ((), ())),
                          preferred_element_type=jnp.float32)
    skip = skip_ref[0]
    a_row = cwt_ref[0:1, :]
    b_row = cwt_ref[1:2, :]
    h2 = skip * (1.0 + a_row) + acc * b_row
    nrm = jnp.maximum(jnp.sqrt(jnp.sum(h2 * h2, axis=1, keepdims=True)), 1e-12)
    hn = h2 / nrm
    r = jnp.maximum(
        lax.dot_general(hn, w1_ref[...], (((1,), (1,)), ((), ())),
                        preferred_element_type=jnp.float32), 0.0)
    pred = jax.nn.sigmoid(
        lax.dot_general(r, w2_ref[...], (((1,), (1,)), ((), ())),
                        preferred_element_type=jnp.float32))
    out_ref[0] = jnp.broadcast_to(pred, (_BL, 16))


def _tc_filter_mlp(h1, cwt, w1, w2):
    hpar = h1.reshape(_M, 2, D).transpose(1, 0, 2)
    hpar_bf = hpar.astype(jnp.bfloat16)
    vstack = jnp.asarray(_VSTACK, dtype=jnp.bfloat16)
    grid = (2, _T)
    out = pl.pallas_call(
        _tc_filter_mlp_body,
        grid=grid,
        in_specs=[
            pl.BlockSpec((1, 2 * _M, _BL), lambda j, i: (j, 0, 0)),
            pl.BlockSpec((1, _M, D), lambda j, i: (1 - j, 0, 0)),
            pl.BlockSpec((1, _BL, D), lambda j, i: (j, i, 0)),
            pl.BlockSpec((2, D), lambda j, i: (0, 0)),
            pl.BlockSpec((D, D), lambda j, i: (0, 0)),
            pl.BlockSpec((1, D), lambda j, i: (0, 0)),
        ],
        out_specs=pl.BlockSpec((1, _BL, 16), lambda j, i: (j, i, 0)),
        out_shape=jax.ShapeDtypeStruct((2, _M, 16), jnp.float32),
    )(vstack, hpar_bf, hpar, cwt, w1, w2)
    return out[:, :, 0].reshape(2 * _M)


def kernel(x, edge_index, edge_label_index, weight1, weight2,
           skip_w0, skip_b0, msg_w0, msg_b0,
           skip_w1, skip_b1, msg_w1, msg_b1, complex_weight):
    pad = jnp.broadcast_to(
        N + jnp.arange(EPW2 - EPW, dtype=jnp.int32), (NW, EPW2 - EPW))
    src3 = jnp.concatenate(
        [edge_index[0].reshape(NW, EPW), pad], axis=1).reshape(NW, NG2, GW)
    dst3 = jnp.concatenate(
        [edge_index[1].reshape(NW, EPW), pad], axis=1).reshape(NW, NG2, GW)
    zrows = jnp.zeros((SLAB, D), jnp.float32)
    ones_in = jnp.ones((GW, D), jnp.float32)

    degp = _sc_degree(src3, zrows, ones_in)
    xs0, dinv = _tc_prep(x, degp)

    agg0 = _sc_aggregate(xs0, src3, dst3, zrows)
    h0, xs1 = _tc_layer(agg0, dinv, x, msg_w0, msg_b0, skip_w0, skip_b0, True)

    agg1 = _sc_aggregate(xs1, src3, dst3, zrows)
    (h1,) = _tc_layer(agg1, dinv, h0, msg_w1, msg_b1, skip_w1, skip_b1, False)

    pred_flat = _tc_filter_mlp(h1, complex_weight.T, weight1, weight2)

    eli0 = jnp.pad(edge_label_index[0], (0, _PPAD - P))
    eli1 = jnp.pad(edge_label_index[1], (0, _PPAD - P))
    prod = _sc_decode(pred_flat, eli0, eli1)
    return prod[:P]
